# baseline (speedup 1.0000x reference)
import functools

import jax
import jax.numpy as jnp
from jax import lax
from jax.experimental import pallas as pl
from jax.experimental.pallas import tpu as pltpu
from jax.experimental.pallas import tpu_sc as plsc

NUM_ENT = 100000
NUM_REL = 1000
ENT_DIM = 128
REL_DIM = 256
BATCH = 16384

NC = 2
NS = 16
NW = NC * NS
BPW = BATCH // NW
CW = 128
NT = 2 * (BPW // CW)
NB = 6

RBLK = 256


def _sc_body(h_idx, t_idx, ent, out_h, out_t, idx_h, idx_t, bufs, gsem, wsem):
    wid = lax.axis_index("s") * NC + lax.axis_index("c")
    base = wid * BPW
    pltpu.sync_copy(h_idx.at[pl.ds(base, BPW)], idx_h)
    pltpu.sync_copy(t_idx.at[pl.ds(base, BPW)], idx_t)

    tasks = []
    for j in range(BPW // CW):
        tasks.append((idx_h.at[pl.ds(j * CW, CW)], out_h, base + j * CW))
        tasks.append((idx_t.at[pl.ds(j * CW, CW)], out_t, base + j * CW))

    def gather(i):
        idx, _, _ = tasks[i]
        b = i % NB
        return pltpu.make_async_copy(ent.at[idx], bufs.at[b], gsem.at[b])

    def write(i):
        _, out, off = tasks[i]
        b = i % NB
        return pltpu.make_async_copy(
            bufs.at[b], out.at[pl.ds(off, CW)], wsem.at[b])

    for i in range(NB):
        gather(i).start()
    waited = set()
    for i in range(NT):
        nk = i + NB - 1
        if i >= 1 and nk < NT:
            write(i - 1).wait()
            waited.add(i - 1)
            gather(nk).start()
        gather(i).wait()
        write(i).start()
    for i in range(NT):
        if i not in waited:
            write(i).wait()


def _tc_body(idx_ref, rel_ref, out_ref, hi_ref, lo_ref):
    @pl.when(pl.program_id(0) == 0)
    def _():
        r = rel_ref[...]
        hi = r.astype(jnp.bfloat16)
        hi_ref[...] = hi
        lo_ref[...] = (r - hi.astype(jnp.float32)).astype(jnp.bfloat16)

    idx_row = idx_ref[0]
    ids = lax.broadcasted_iota(jnp.int32, (NUM_REL, RBLK), 0)
    onehot = (ids == idx_row).astype(jnp.bfloat16)
    dn = (((0,), (0,)), ((), ()))
    acc = lax.dot_general(onehot, hi_ref[...], dn,
                          preferred_element_type=jnp.float32)
    acc = acc + lax.dot_general(onehot, lo_ref[...], dn,
                                preferred_element_type=jnp.float32)
    out_ref[...] = acc


@jax.jit
def _gather3(h_idx, t_idx, r_idx3, ent_table, rel_table):
    mesh = plsc.VectorSubcoreMesh(core_axis_name="c", subcore_axis_name="s")
    sc = pl.kernel(
        _sc_body,
        out_type=(
            jax.ShapeDtypeStruct((BATCH, ENT_DIM), jnp.float32),
            jax.ShapeDtypeStruct((BATCH, ENT_DIM), jnp.float32),
        ),
        mesh=mesh,
        scratch_types=[
            pltpu.VMEM((BPW,), jnp.int32),
            pltpu.VMEM((BPW,), jnp.int32),
            pltpu.VMEM((NB, CW, ENT_DIM), jnp.float32),
            pltpu.SemaphoreType.DMA((NB,)),
            pltpu.SemaphoreType.DMA((NB,)),
        ],
    )
    out_h, out_t = sc(h_idx, t_idx, ent_table)

    out_r = pl.pallas_call(
        _tc_body,
        grid=(BATCH // RBLK,),
        in_specs=[
            pl.BlockSpec((1, 1, RBLK), lambda i: (i, 0, 0)),
            pl.BlockSpec((NUM_REL, REL_DIM), lambda i: (0, 0)),
        ],
        out_specs=pl.BlockSpec((RBLK, REL_DIM), lambda i: (i, 0)),
        out_shape=jax.ShapeDtypeStruct((BATCH, REL_DIM), jnp.float32),
        scratch_shapes=[
            pltpu.VMEM((NUM_REL, REL_DIM), jnp.bfloat16),
            pltpu.VMEM((NUM_REL, REL_DIM), jnp.bfloat16),
        ],
    )(r_idx3, rel_table)
    return out_h, out_t, out_r


def kernel(batch_h, batch_t, batch_r, mode, ent_table, rel_table):
    del mode
    r3 = batch_r.reshape(BATCH // RBLK, 1, RBLK)
    return _gather3(batch_h, batch_t, r3, ent_table, rel_table)
